# baseline (speedup 1.0000x reference)
import functools

import jax
import jax.numpy as jnp
from jax import lax
from jax.experimental import pallas as pl
from jax.experimental.pallas import tpu as pltpu
from jax.experimental.pallas import tpu_sc as plsc

VOCAB = 1000000
EMBED = 64
SEQ = 200
BATCH = 1024
NW = 32
BPW = BATCH // NW
LANES = 16
NBUF = 4
QUADS = BPW // NBUF

RB = 128
NBLK = VOCAB // RB
BLK_PER_W = NBLK // NW
EXTRA = NBLK - NW * BLK_PER_W
TAIL = VOCAB - NBLK * RB


def _tr_body(tt_hbm, tail_hbm, out_hbm, src0, src1, dst0, dst1,
             gsem0, gsem1, ssem0, ssem1):
    c = lax.axis_index("c")
    s = lax.axis_index("s")
    wid = s * 2 + c
    g0 = wid * BLK_PER_W

    srcs = (src0, src1)
    dsts = (dst0, dst1)
    gsems = (gsem0, gsem1)
    ssems = (ssem0, ssem1)

    def start_load(g, b):
        pltpu.async_copy(tt_hbm.at[:, pl.ds(g * RB, RB)], srcs[b], gsems[b])

    def wait_load(b):
        pltpu.make_async_copy(tt_hbm.at[:, pl.ds(0, RB)], srcs[b], gsems[b]).wait()

    def start_store(g, b):
        pltpu.async_copy(dsts[b], out_hbm.at[pl.ds(g * (RB // 2), RB // 2)], ssems[b])

    def wait_store(b):
        pltpu.make_async_copy(dsts[b], out_hbm.at[pl.ds(0, RB // 2)], ssems[b]).wait()

    lanes = jnp.arange(LANES, dtype=jnp.int32)

    def transpose(b, nrows):
        src = srcs[b]
        dst = dsts[b]

        def row_pair(rp, c2):
            for rr in range(2):
                r = 2 * rp + rr
                rid = jnp.full((LANES,), 0, dtype=jnp.int32) + r
                for k in range(EMBED // LANES):
                    v = plsc.load_gather(src, [k * LANES + lanes, rid])
                    dst[rp, pl.ds(rr * EMBED + k * LANES, LANES)] = v
            return c2

        lax.fori_loop(0, nrows // 2, row_pair, 0, unroll=8)

    start_load(g0, 0)

    def pair(p, c2):
        for par in range(2):
            t = 2 * p + par
            @pl.when(t + 1 < BLK_PER_W)
            def _():
                start_load(g0 + t + 1, 1 - par)

            @pl.when(t >= 2)
            def _():
                wait_store(par)

            wait_load(par)
            transpose(par, RB)
            start_store(g0 + t, par)
        return c2

    lax.fori_loop(0, BLK_PER_W // 2, pair, 0)
    wait_store(0)
    wait_store(1)

    @pl.when(wid < EXTRA)
    def _():
        g = NW * BLK_PER_W + wid
        start_load(g, 0)
        wait_load(0)
        transpose(0, RB)
        pltpu.sync_copy(dsts[0], out_hbm.at[pl.ds(g * (RB // 2), RB // 2)])

    @pl.when(wid == EXTRA)
    def _():
        pltpu.sync_copy(tail_hbm, dsts[0].at[pl.ds(0, TAIL // 2)])
        pltpu.sync_copy(
            dsts[0].at[pl.ds(0, TAIL // 2)],
            out_hbm.at[pl.ds(NBLK * (RB // 2), TAIL // 2)],
        )


@jax.jit
def _transpose_table(tt, tail):
    mesh = plsc.VectorSubcoreMesh(core_axis_name="c", subcore_axis_name="s")
    f = functools.partial(
        pl.kernel,
        mesh=mesh,
        out_type=jax.ShapeDtypeStruct((VOCAB // 2, 2 * EMBED), jnp.float32),
        scratch_types=[
            pltpu.VMEM((EMBED, RB), jnp.float32),
            pltpu.VMEM((EMBED, RB), jnp.float32),
            pltpu.VMEM((RB // 2, 2 * EMBED), jnp.float32),
            pltpu.VMEM((RB // 2, 2 * EMBED), jnp.float32),
            pltpu.SemaphoreType.DMA,
            pltpu.SemaphoreType.DMA,
            pltpu.SemaphoreType.DMA,
            pltpu.SemaphoreType.DMA,
        ],
        compiler_params=pltpu.CompilerParams(
            use_tc_tiling_on_sc=True, needs_layout_passes=False
        ),
    )(_tr_body)
    return f(tt, tail)


def _emb_body(ids_hbm, table_hbm, pos_hbm, out_hbm, idx_v, pos_v, bufs, gsems, ssems):
    c = lax.axis_index("c")
    s = lax.axis_index("s")
    wid = s * 2 + c
    b0 = wid * BPW

    pltpu.sync_copy(ids_hbm.at[pl.ds(b0, BPW)], idx_v)
    pltpu.sync_copy(pos_hbm.at[pl.ds(0, SEQ)], pos_v)

    def start_gather(bl, b):
        for off, n in ((0, 104), (104, 96)):
            pltpu.async_copy(
                table_hbm.at[idx_v.at[bl, pl.ds(off, n)]],
                bufs[b].at[pl.ds(off, n)],
                gsems[b],
            )

    def wait_gather(b):
        pltpu.make_async_copy(
            table_hbm.at[idx_v.at[0, pl.ds(0, 104)]], bufs[b], gsems[b]
        ).wait()

    def start_store(bl, b):
        pltpu.async_copy(bufs[b], out_hbm.at[b0 + bl], ssems[b])

    def wait_store(b):
        pltpu.make_async_copy(bufs[b], out_hbm.at[0], ssems[b]).wait()

    def add_pos(b):
        buf = bufs[b]

        def add_row(j, c2):
            for jj in range(2):
                for k in range(EMBED // LANES):
                    sl = pl.ds(k * LANES, LANES)
                    plsc.addupdate(buf.at[2 * j + jj, sl], pos_v[2 * j + jj, sl])
            return c2

        lax.fori_loop(0, SEQ // 2, add_row, 0)

    start_gather(0, 0)
    start_gather(1, 1)

    def quad(q, carry):
        for i in range(NBUF):
            bl = NBUF * q + i
            b2 = (i + 2) % NBUF
            if i < 2:
                @pl.when(q >= 1)
                def _():
                    wait_store(b2)
                    start_gather(bl + 2, b2)

                @pl.when(q < 1)
                def _():
                    start_gather(bl + 2, b2)
            else:
                wait_store(b2)

                @pl.when(q < QUADS - 1)
                def _():
                    start_gather(bl + 2, b2)

            wait_gather(i)
            add_pos(i)
            start_store(bl, i)
        return carry

    lax.fori_loop(0, QUADS, quad, 0)
    wait_store(2)
    wait_store(3)


@jax.jit
def _emb(ids, table, pos):
    mesh = plsc.VectorSubcoreMesh(core_axis_name="c", subcore_axis_name="s")
    f = functools.partial(
        pl.kernel,
        mesh=mesh,
        out_type=jax.ShapeDtypeStruct((BATCH, SEQ, EMBED), jnp.float32),
        scratch_types=[
            pltpu.VMEM((BPW, SEQ), jnp.int32),
            pltpu.VMEM((SEQ, EMBED), jnp.float32),
            [pltpu.VMEM((SEQ, EMBED), jnp.float32) for _ in range(NBUF)],
            [pltpu.SemaphoreType.DMA for _ in range(NBUF)],
            [pltpu.SemaphoreType.DMA for _ in range(NBUF)],
        ],
        compiler_params=pltpu.CompilerParams(use_tc_tiling_on_sc=False),
    )(_emb_body)
    return f(ids, table, pos)


def kernel(token_ids, token_table, pos_table):
    tail = token_table[NBLK * RB:].reshape(TAIL // 2, 2 * EMBED)
    lin = _transpose_table(token_table.T, tail)
    table_lin = lin.reshape(VOCAB, EMBED)
    return _emb(token_ids, table_lin, pos_table)


# speedup vs baseline: 1.0006x; 1.0006x over previous
import functools

import jax
import jax.numpy as jnp
from jax import lax
from jax.experimental import pallas as pl
from jax.experimental.pallas import tpu as pltpu
from jax.experimental.pallas import tpu_sc as plsc

VOCAB = 1000000
EMBED = 64
SEQ = 200
BATCH = 1024
NW = 32
BPW = BATCH // NW
LANES = 16
NBUF = 4
QUADS = BPW // NBUF

RB = 128
NBLK = VOCAB // RB
BLK_PER_W = NBLK // NW
EXTRA = NBLK - NW * BLK_PER_W
TAIL = VOCAB - NBLK * RB


def _tr_body(tt_hbm, tail_hbm, out_hbm, src0, src1, dst0, dst1,
             gsem0, gsem1, ssem0, ssem1):
    c = lax.axis_index("c")
    s = lax.axis_index("s")
    wid = s * 2 + c
    g0 = wid * BLK_PER_W

    srcs = (src0, src1)
    dsts = (dst0, dst1)
    gsems = (gsem0, gsem1)
    ssems = (ssem0, ssem1)

    def start_load(g, b):
        pltpu.async_copy(
            tt_hbm.at[:, pl.ds(g * RB, RB)], srcs[b].at[:, pl.ds(0, RB)], gsems[b]
        )

    def wait_load(b):
        pltpu.make_async_copy(
            tt_hbm.at[:, pl.ds(0, RB)], srcs[b].at[:, pl.ds(0, RB)], gsems[b]
        ).wait()

    def start_store(g, b):
        pltpu.async_copy(dsts[b], out_hbm.at[pl.ds(g * (RB // 2), RB // 2)], ssems[b])

    def wait_store(b):
        pltpu.make_async_copy(dsts[b], out_hbm.at[pl.ds(0, RB // 2)], ssems[b]).wait()

    lanes = jnp.arange(LANES, dtype=jnp.int32)

    def transpose(b, nrows):
        src = srcs[b]
        dst = dsts[b]

        def row_pair(rp, c2):
            for rr in range(2):
                r = 2 * rp + rr
                rid = jnp.full((LANES,), 0, dtype=jnp.int32) + r
                for k in range(EMBED // LANES):
                    v = plsc.load_gather(src, [k * LANES + lanes, rid])
                    dst[rp, pl.ds(rr * EMBED + k * LANES, LANES)] = v
            return c2

        lax.fori_loop(0, nrows // 2, row_pair, 0, unroll=8)

    start_load(g0, 0)

    def pair(p, c2):
        for par in range(2):
            t = 2 * p + par
            @pl.when(t + 1 < BLK_PER_W)
            def _():
                start_load(g0 + t + 1, 1 - par)

            @pl.when(t >= 2)
            def _():
                wait_store(par)

            wait_load(par)
            transpose(par, RB)
            start_store(g0 + t, par)
        return c2

    lax.fori_loop(0, BLK_PER_W // 2, pair, 0)
    wait_store(0)
    wait_store(1)

    @pl.when(wid < EXTRA)
    def _():
        g = NW * BLK_PER_W + wid
        start_load(g, 0)
        wait_load(0)
        transpose(0, RB)
        pltpu.sync_copy(dsts[0], out_hbm.at[pl.ds(g * (RB // 2), RB // 2)])

    @pl.when(wid == EXTRA)
    def _():
        pltpu.sync_copy(tail_hbm, dsts[0].at[pl.ds(0, TAIL // 2)])
        pltpu.sync_copy(
            dsts[0].at[pl.ds(0, TAIL // 2)],
            out_hbm.at[pl.ds(NBLK * (RB // 2), TAIL // 2)],
        )


@jax.jit
def _transpose_table(tt, tail):
    mesh = plsc.VectorSubcoreMesh(core_axis_name="c", subcore_axis_name="s")
    f = functools.partial(
        pl.kernel,
        mesh=mesh,
        out_type=jax.ShapeDtypeStruct((VOCAB // 2, 2 * EMBED), jnp.float32),
        scratch_types=[
            pltpu.VMEM((EMBED, RB + 9), jnp.float32),
            pltpu.VMEM((EMBED, RB + 9), jnp.float32),
            pltpu.VMEM((RB // 2, 2 * EMBED), jnp.float32),
            pltpu.VMEM((RB // 2, 2 * EMBED), jnp.float32),
            pltpu.SemaphoreType.DMA,
            pltpu.SemaphoreType.DMA,
            pltpu.SemaphoreType.DMA,
            pltpu.SemaphoreType.DMA,
        ],
        compiler_params=pltpu.CompilerParams(
            use_tc_tiling_on_sc=True, needs_layout_passes=False
        ),
    )(_tr_body)
    return f(tt, tail)


def _emb_body(ids_hbm, table_hbm, pos_hbm, out_hbm, idx_v, pos_v, bufs, gsems, ssems):
    c = lax.axis_index("c")
    s = lax.axis_index("s")
    wid = s * 2 + c
    b0 = wid * BPW

    pltpu.sync_copy(ids_hbm.at[pl.ds(b0, BPW)], idx_v)
    pltpu.sync_copy(pos_hbm.at[pl.ds(0, SEQ)], pos_v)

    def start_gather(bl, b):
        for off, n in ((0, 104), (104, 96)):
            pltpu.async_copy(
                table_hbm.at[idx_v.at[bl, pl.ds(off, n)]],
                bufs[b].at[pl.ds(off, n)],
                gsems[b],
            )

    def wait_gather(b):
        pltpu.make_async_copy(
            table_hbm.at[idx_v.at[0, pl.ds(0, 104)]], bufs[b], gsems[b]
        ).wait()

    def start_store(bl, b):
        pltpu.async_copy(bufs[b], out_hbm.at[b0 + bl], ssems[b])

    def wait_store(b):
        pltpu.make_async_copy(bufs[b], out_hbm.at[0], ssems[b]).wait()

    def add_pos(b):
        buf = bufs[b]

        def add_row(j, c2):
            for jj in range(2):
                for k in range(EMBED // LANES):
                    sl = pl.ds(k * LANES, LANES)
                    plsc.addupdate(buf.at[2 * j + jj, sl], pos_v[2 * j + jj, sl])
            return c2

        lax.fori_loop(0, SEQ // 2, add_row, 0)

    start_gather(0, 0)
    start_gather(1, 1)

    def quad(q, carry):
        for i in range(NBUF):
            bl = NBUF * q + i
            b2 = (i + 2) % NBUF
            if i < 2:
                @pl.when(q >= 1)
                def _():
                    wait_store(b2)
                    start_gather(bl + 2, b2)

                @pl.when(q < 1)
                def _():
                    start_gather(bl + 2, b2)
            else:
                wait_store(b2)

                @pl.when(q < QUADS - 1)
                def _():
                    start_gather(bl + 2, b2)

            wait_gather(i)
            add_pos(i)
            start_store(bl, i)
        return carry

    lax.fori_loop(0, QUADS, quad, 0)
    wait_store(2)
    wait_store(3)


@jax.jit
def _emb(ids, table, pos):
    mesh = plsc.VectorSubcoreMesh(core_axis_name="c", subcore_axis_name="s")
    f = functools.partial(
        pl.kernel,
        mesh=mesh,
        out_type=jax.ShapeDtypeStruct((BATCH, SEQ, EMBED), jnp.float32),
        scratch_types=[
            pltpu.VMEM((BPW, SEQ), jnp.int32),
            pltpu.VMEM((SEQ, EMBED), jnp.float32),
            [pltpu.VMEM((SEQ, EMBED), jnp.float32) for _ in range(NBUF)],
            [pltpu.SemaphoreType.DMA for _ in range(NBUF)],
            [pltpu.SemaphoreType.DMA for _ in range(NBUF)],
        ],
        compiler_params=pltpu.CompilerParams(use_tc_tiling_on_sc=False),
    )(_emb_body)
    return f(ids, table, pos)


def kernel(token_ids, token_table, pos_table):
    tail = token_table[NBLK * RB:].reshape(TAIL // 2, 2 * EMBED)
    lin = _transpose_table(token_table.T, tail)
    table_lin = lin.reshape(VOCAB, EMBED)
    return _emb(token_ids, table_lin, pos_table)


# speedup vs baseline: 4.4620x; 4.4592x over previous
import functools

import jax
import jax.numpy as jnp
from jax import lax
from jax.experimental import pallas as pl
from jax.experimental.pallas import tpu as pltpu
from jax.experimental.pallas import tpu_sc as plsc

VOCAB = 1000000
EMBED = 64
SEQ = 200
BATCH = 1024
NW = 32
BPW = BATCH // NW
LANES = 16
NBUF = 4
QUADS = BPW // NBUF

RB = 128
NBLK = VOCAB // RB
BLK_PER_W = NBLK // NW
EXTRA = NBLK - NW * BLK_PER_W
TAIL = VOCAB - NBLK * RB


def _tr_body(tt_hbm, tail_hbm, out_hbm, src0, src1, dst0, dst1,
             gsem0, gsem1, ssem0, ssem1):
    c = lax.axis_index("c")
    s = lax.axis_index("s")
    wid = s * 2 + c
    g0 = wid * BLK_PER_W

    srcs = (src0, src1)
    dsts = (dst0, dst1)
    gsems = (gsem0, gsem1)
    ssems = (ssem0, ssem1)

    def start_load(g, b):
        pltpu.async_copy(
            tt_hbm.at[:, pl.ds(g * RB, RB)], srcs[b].at[:, pl.ds(0, RB)], gsems[b]
        )

    def wait_load(b):
        pltpu.make_async_copy(
            tt_hbm.at[:, pl.ds(0, RB)], srcs[b].at[:, pl.ds(0, RB)], gsems[b]
        ).wait()

    def start_store(g, b):
        pltpu.async_copy(dsts[b], out_hbm.at[pl.ds(g * (RB // 2), RB // 2)], ssems[b])

    def wait_store(b):
        pltpu.make_async_copy(dsts[b], out_hbm.at[pl.ds(0, RB // 2)], ssems[b]).wait()

    lanes = jnp.arange(LANES, dtype=jnp.int32)

    def transpose(b, nrows):
        src = srcs[b]
        dst = dsts[b]

        @functools.partial(plsc.parallel_loop, 0, nrows // 2, unroll=4)
        def _(rp):
            for rr in range(2):
                r = 2 * rp + rr
                rid = jnp.full((LANES,), 0, dtype=jnp.int32) + r
                for k in range(EMBED // LANES):
                    v = plsc.load_gather(src, [k * LANES + lanes, rid])
                    dst[rp, pl.ds(rr * EMBED + k * LANES, LANES)] = v

    start_load(g0, 0)

    def pair(p, c2):
        for par in range(2):
            t = 2 * p + par
            @pl.when(t + 1 < BLK_PER_W)
            def _():
                start_load(g0 + t + 1, 1 - par)

            @pl.when(t >= 2)
            def _():
                wait_store(par)

            wait_load(par)
            transpose(par, RB)
            start_store(g0 + t, par)
        return c2

    lax.fori_loop(0, BLK_PER_W // 2, pair, 0)
    wait_store(0)
    wait_store(1)

    @pl.when(wid < EXTRA)
    def _():
        g = NW * BLK_PER_W + wid
        start_load(g, 0)
        wait_load(0)
        transpose(0, RB)
        pltpu.sync_copy(dsts[0], out_hbm.at[pl.ds(g * (RB // 2), RB // 2)])

    @pl.when(wid == EXTRA)
    def _():
        pltpu.sync_copy(tail_hbm, dsts[0].at[pl.ds(0, TAIL // 2)])
        pltpu.sync_copy(
            dsts[0].at[pl.ds(0, TAIL // 2)],
            out_hbm.at[pl.ds(NBLK * (RB // 2), TAIL // 2)],
        )


@jax.jit
def _transpose_table(tt, tail):
    mesh = plsc.VectorSubcoreMesh(core_axis_name="c", subcore_axis_name="s")
    f = functools.partial(
        pl.kernel,
        mesh=mesh,
        out_type=jax.ShapeDtypeStruct((VOCAB // 2, 2 * EMBED), jnp.float32),
        scratch_types=[
            pltpu.VMEM((EMBED, RB + 9), jnp.float32),
            pltpu.VMEM((EMBED, RB + 9), jnp.float32),
            pltpu.VMEM((RB // 2, 2 * EMBED), jnp.float32),
            pltpu.VMEM((RB // 2, 2 * EMBED), jnp.float32),
            pltpu.SemaphoreType.DMA,
            pltpu.SemaphoreType.DMA,
            pltpu.SemaphoreType.DMA,
            pltpu.SemaphoreType.DMA,
        ],
        compiler_params=pltpu.CompilerParams(
            use_tc_tiling_on_sc=True, needs_layout_passes=False
        ),
    )(_tr_body)
    return f(tt, tail)


def _emb_body(ids_hbm, table_hbm, pos_hbm, out_hbm, idx_v, pos_v, bufs, gsems, ssems):
    c = lax.axis_index("c")
    s = lax.axis_index("s")
    wid = s * 2 + c
    b0 = wid * BPW

    pltpu.sync_copy(ids_hbm.at[pl.ds(b0, BPW)], idx_v)
    pltpu.sync_copy(pos_hbm.at[pl.ds(0, SEQ)], pos_v)

    def start_gather(bl, b):
        for off, n in ((0, 104), (104, 96)):
            pltpu.async_copy(
                table_hbm.at[idx_v.at[bl, pl.ds(off, n)]],
                bufs[b].at[pl.ds(off, n)],
                gsems[b],
            )

    def wait_gather(b):
        pltpu.make_async_copy(
            table_hbm.at[idx_v.at[0, pl.ds(0, 104)]], bufs[b], gsems[b]
        ).wait()

    def start_store(bl, b):
        pltpu.async_copy(bufs[b], out_hbm.at[b0 + bl], ssems[b])

    def wait_store(b):
        pltpu.make_async_copy(bufs[b], out_hbm.at[0], ssems[b]).wait()

    def add_pos(b):
        buf = bufs[b]

        def add_row(j, c2):
            for jj in range(2):
                for k in range(EMBED // LANES):
                    sl = pl.ds(k * LANES, LANES)
                    plsc.addupdate(buf.at[2 * j + jj, sl], pos_v[2 * j + jj, sl])
            return c2

        lax.fori_loop(0, SEQ // 2, add_row, 0)

    start_gather(0, 0)
    start_gather(1, 1)

    def quad(q, carry):
        for i in range(NBUF):
            bl = NBUF * q + i
            b2 = (i + 2) % NBUF
            if i < 2:
                @pl.when(q >= 1)
                def _():
                    wait_store(b2)
                    start_gather(bl + 2, b2)

                @pl.when(q < 1)
                def _():
                    start_gather(bl + 2, b2)
            else:
                wait_store(b2)

                @pl.when(q < QUADS - 1)
                def _():
                    start_gather(bl + 2, b2)

            wait_gather(i)
            add_pos(i)
            start_store(bl, i)
        return carry

    lax.fori_loop(0, QUADS, quad, 0)
    wait_store(2)
    wait_store(3)


@jax.jit
def _emb(ids, table, pos):
    mesh = plsc.VectorSubcoreMesh(core_axis_name="c", subcore_axis_name="s")
    f = functools.partial(
        pl.kernel,
        mesh=mesh,
        out_type=jax.ShapeDtypeStruct((BATCH, SEQ, EMBED), jnp.float32),
        scratch_types=[
            pltpu.VMEM((BPW, SEQ), jnp.int32),
            pltpu.VMEM((SEQ, EMBED), jnp.float32),
            [pltpu.VMEM((SEQ, EMBED), jnp.float32) for _ in range(NBUF)],
            [pltpu.SemaphoreType.DMA for _ in range(NBUF)],
            [pltpu.SemaphoreType.DMA for _ in range(NBUF)],
        ],
        compiler_params=pltpu.CompilerParams(use_tc_tiling_on_sc=False),
    )(_emb_body)
    return f(ids, table, pos)


def kernel(token_ids, token_table, pos_table):
    tail = token_table[NBLK * RB:].reshape(TAIL // 2, 2 * EMBED)
    lin = _transpose_table(token_table.T, tail)
    table_lin = lin.reshape(VOCAB, EMBED)
    return _emb(token_ids, table_lin, pos_table)
